# Initial kernel scaffold; baseline (speedup 1.0000x reference)
import jax
import jax.numpy as jnp
from jax.experimental import pallas as pl


def kernel(x, emb_table):
    raise NotImplementedError("write your pallas kernel here")


# speedup vs baseline: 1.3781x; 1.3781x over previous
import functools

import jax
import jax.numpy as jnp
from jax import lax
from jax.experimental import pallas as pl
from jax.experimental.pallas import tpu as pltpu
from jax.experimental.pallas import tpu_sc as plsc

_MAX_LEN = 8192
_HIDDEN = 128

_INFO = plsc.get_sparse_core_info()
_NC = _INFO.num_cores
_NS = _INFO.num_subcores
_NW = _NC * _NS
_ROWS_PER_W = _MAX_LEN // _NW


def _make_copy():
    mesh = plsc.VectorSubcoreMesh(core_axis_name="c", subcore_axis_name="s")

    @functools.partial(
        pl.kernel,
        mesh=mesh,
        out_type=jax.ShapeDtypeStruct((_MAX_LEN, _HIDDEN), jnp.float32),
        scratch_types=[pltpu.VMEM((_ROWS_PER_W, _HIDDEN), jnp.float32)],
    )
    def k(table_hbm, out_hbm, buf):
        wid = lax.axis_index("s") * _NC + lax.axis_index("c")
        base = wid * _ROWS_PER_W
        pltpu.sync_copy(table_hbm.at[pl.ds(base, _ROWS_PER_W)], buf)
        pltpu.sync_copy(buf, out_hbm.at[pl.ds(base, _ROWS_PER_W)])

    return k


_copy = _make_copy()


def kernel(x, emb_table):
    seq_len = x.shape[1]
    out = _copy(emb_table)
    return out[None, :seq_len]
